# Initial kernel scaffold; baseline (speedup 1.0000x reference)
import functools

import jax
import jax.numpy as jnp
from jax import lax
from jax.experimental import pallas as pl
from jax.experimental.pallas import tpu as pltpu
from jax.experimental.pallas import tpu_sc as plsc

E = 16
TOPK = 2
H = 1024
FDIM = 1024
D = 2 * FDIM
R = 4
SCALING = 1.0 / R
LIMIT = 7.0
ACT_ALPHA = 1.702
T = 4096
P = T * TOPK

BM = 128
NB = P // BM + E
PM = NB * BM

NC, NS, NL = 2, 16, 16
NW = NC * NS

GPW = PM // NW
GCH = 64


def _sc_gather_body(x_hbm, tok_hbm, xs_hbm, idx_v, buf_v, sem):
    wid = lax.axis_index("s") * NC + lax.axis_index("c")
    base = wid * GPW
    pltpu.sync_copy(tok_hbm.at[pl.ds(base, GPW)], idx_v)
    for c in range(GPW // GCH):
        pltpu.async_copy(
            x_hbm.at[idx_v.at[pl.ds(c * GCH, GCH)]], buf_v, sem).wait()
        pltpu.sync_copy(buf_v, xs_hbm.at[pl.ds(base + c * GCH, GCH)])


_sc_gather = functools.partial(
    pl.kernel,
    out_type=jax.ShapeDtypeStruct((PM, H), jnp.float32),
    mesh=plsc.VectorSubcoreMesh(core_axis_name="c", subcore_axis_name="s"),
    scratch_types=[
        pltpu.VMEM((GPW,), jnp.int32),
        pltpu.VMEM((GCH, H), jnp.float32),
        pltpu.SemaphoreType.DMA,
    ],
)(_sc_gather_body)


TPW = T // NW
CCH = 32


def _sc_combine_body(outs_hbm, posa_hbm, posb_hbm, y_hbm,
                     idxa_v, idxb_v, bufa_v, bufb_v, sema, semb):
    wid = lax.axis_index("s") * NC + lax.axis_index("c")
    base = wid * TPW
    pltpu.sync_copy(posa_hbm.at[pl.ds(base, TPW)], idxa_v)
    pltpu.sync_copy(posb_hbm.at[pl.ds(base, TPW)], idxb_v)
    for c in range(TPW // CCH):
        cpA = pltpu.async_copy(
            outs_hbm.at[idxa_v.at[pl.ds(c * CCH, CCH)]], bufa_v, sema)
        cpB = pltpu.async_copy(
            outs_hbm.at[idxb_v.at[pl.ds(c * CCH, CCH)]], bufb_v, semb)
        cpA.wait()
        cpB.wait()
        for r in range(CCH):
            def add_row(i, carry, r=r):
                sl = pl.ds(i * NL, NL)
                bufa_v[r, sl] = bufa_v[r, sl] + bufb_v[r, sl]
                return carry
            lax.fori_loop(0, H // NL, add_row, 0)
        pltpu.sync_copy(bufa_v, y_hbm.at[pl.ds(base + c * CCH, CCH)])


_sc_combine = functools.partial(
    pl.kernel,
    out_type=jax.ShapeDtypeStruct((T, H), jnp.float32),
    mesh=plsc.VectorSubcoreMesh(core_axis_name="c", subcore_axis_name="s"),
    scratch_types=[
        pltpu.VMEM((TPW,), jnp.int32),
        pltpu.VMEM((TPW,), jnp.int32),
        pltpu.VMEM((CCH, H), jnp.float32),
        pltpu.VMEM((CCH, H), jnp.float32),
        pltpu.SemaphoreType.DMA,
        pltpu.SemaphoreType.DMA,
    ],
)(_sc_combine_body)


def _ffn_body(be_ref, xs_ref, w1_ref, b1_ref, a1_ref, bb1_ref,
              w2_ref, b2_ref, a2_ref, bb2_ref, wrow_ref, out_ref):
    xb = xs_ref[...]
    gu = jnp.dot(xb, w1_ref[0], preferred_element_type=jnp.float32)
    lo = jnp.dot(xb, a1_ref[0], preferred_element_type=jnp.float32)
    gu = gu + jnp.dot(lo, bb1_ref[0],
                      preferred_element_type=jnp.float32) * SCALING
    gu = gu + b1_ref[...]
    gate = gu[:, 0::2]
    up = gu[:, 1::2]
    gate = jnp.minimum(gate, LIMIT)
    up = jnp.clip(up, -LIMIT, LIMIT)
    glu = gate * jax.nn.sigmoid(gate * ACT_ALPHA)
    gated = (up + 1.0) * glu
    dn = jnp.dot(gated, w2_ref[0], preferred_element_type=jnp.float32)
    lo2 = jnp.dot(gated, a2_ref[0], preferred_element_type=jnp.float32)
    dn = dn + jnp.dot(lo2, bb2_ref[0],
                      preferred_element_type=jnp.float32) * SCALING
    dn = dn + b2_ref[...]
    out_ref[...] = dn * wrow_ref[...]


def _ffn(xs, w1, b1, a1, bb1, w2, b2, a2, bb2, wrow, block_expert):
    def xmap(i, be):
        return (i, 0)

    def emap(i, be):
        return (be[i], 0, 0)

    def emap2(i, be):
        return (be[i], 0)

    grid_spec = pltpu.PrefetchScalarGridSpec(
        num_scalar_prefetch=1,
        grid=(NB,),
        in_specs=[
            pl.BlockSpec((BM, H), xmap),
            pl.BlockSpec((1, H, D), emap),
            pl.BlockSpec((1, D), emap2),
            pl.BlockSpec((1, H, R), emap),
            pl.BlockSpec((1, R, D), emap),
            pl.BlockSpec((1, FDIM, H), emap),
            pl.BlockSpec((1, H), emap2),
            pl.BlockSpec((1, FDIM, R), emap),
            pl.BlockSpec((1, R, H), emap),
            pl.BlockSpec((BM, 1), xmap),
        ],
        out_specs=pl.BlockSpec((BM, H), xmap),
    )
    return pl.pallas_call(
        _ffn_body,
        grid_spec=grid_spec,
        out_shape=jax.ShapeDtypeStruct((PM, H), jnp.float32),
    )(block_expert, xs, w1, b1, a1, bb1, w2, b2, a2, bb2, wrow)


def kernel(hidden_states, router_indices, routing_weights, gate_up_proj,
           gate_up_proj_bias, down_proj, down_proj_bias, lora_gate_up_A,
           lora_gate_up_B, lora_down_A, lora_down_B):
    batch_size = hidden_states.shape[0]
    x = hidden_states.reshape(T, H)

    ri = router_indices.reshape(-1).astype(jnp.int32)
    rw = routing_weights.reshape(-1).astype(jnp.float32)
    perm = jnp.argsort(ri).astype(jnp.int32)
    se = jnp.take(ri, perm)
    counts = jnp.zeros((E,), jnp.int32).at[ri].add(1)
    nblk = (counts + BM - 1) // BM
    pad_off = jnp.concatenate(
        [jnp.zeros((1,), jnp.int32), jnp.cumsum(nblk * BM)[:-1]])
    tight_off = jnp.concatenate(
        [jnp.zeros((1,), jnp.int32), jnp.cumsum(counts)[:-1]])
    ranks = jnp.arange(P, dtype=jnp.int32) - jnp.take(tight_off, se)
    slot = jnp.take(pad_off, se) + ranks
    tok_for_slot = jnp.zeros((PM,), jnp.int32).at[slot].set(
        (perm // TOPK).astype(jnp.int32))
    w_for_slot = jnp.zeros((PM,), jnp.float32).at[slot].set(jnp.take(rw, perm))
    pos = jnp.zeros((P,), jnp.int32).at[perm].set(slot)
    posA = pos[0::2]
    posB = pos[1::2]
    block_expert = jnp.minimum(
        jnp.searchsorted(jnp.cumsum(nblk), jnp.arange(NB), side="right"),
        E - 1).astype(jnp.int32)

    xs = _sc_gather(x, tok_for_slot)
    outs = _ffn(xs, gate_up_proj, gate_up_proj_bias, lora_gate_up_A,
                lora_gate_up_B, down_proj, down_proj_bias, lora_down_A,
                lora_down_B, w_for_slot.reshape(PM, 1), block_expert)
    y = _sc_combine(outs, posA, posB)
    return y.reshape(batch_size, -1, H)


# speedup vs baseline: 3.8386x; 3.8386x over previous
import functools

import jax
import jax.numpy as jnp
from jax import lax
from jax.experimental import pallas as pl
from jax.experimental.pallas import tpu as pltpu
from jax.experimental.pallas import tpu_sc as plsc

E = 16
TOPK = 2
H = 1024
FDIM = 1024
D = 2 * FDIM
R = 4
SCALING = 1.0 / R
LIMIT = 7.0
ACT_ALPHA = 1.702
T = 4096
P = T * TOPK

BM = 128
NB = P // BM + E
PM = NB * BM

NC, NS, NL = 2, 16, 16
NW = NC * NS

GPW = PM // NW
GCH = 64


def _sc_gather_body(x_hbm, tok_hbm, xs_hbm, idx_v, buf_v, sem):
    wid = lax.axis_index("s") * NC + lax.axis_index("c")
    base = wid * GPW
    pltpu.sync_copy(tok_hbm.at[pl.ds(base, GPW)], idx_v)
    for c in range(GPW // GCH):
        pltpu.async_copy(
            x_hbm.at[idx_v.at[pl.ds(c * GCH, GCH)]], buf_v, sem).wait()
        pltpu.sync_copy(buf_v, xs_hbm.at[pl.ds(base + c * GCH, GCH)])


@functools.cache
def _sc_gather():
    return functools.partial(
        pl.kernel,
        out_type=jax.ShapeDtypeStruct((PM, H), jnp.float32),
        mesh=plsc.VectorSubcoreMesh(core_axis_name="c", subcore_axis_name="s"),
        scratch_types=[
            pltpu.VMEM((GPW,), jnp.int32),
            pltpu.VMEM((GCH, H), jnp.float32),
            pltpu.SemaphoreType.DMA,
        ],
    )(_sc_gather_body)


TPW = T // NW
CCH = 32


def _sc_combine_body(outs_hbm, posa_hbm, posb_hbm, y_hbm,
                     idxa_v, idxb_v, bufa_v, bufb_v, sema, semb):
    wid = lax.axis_index("s") * NC + lax.axis_index("c")
    base = wid * TPW
    pltpu.sync_copy(posa_hbm.at[pl.ds(base, TPW)], idxa_v)
    pltpu.sync_copy(posb_hbm.at[pl.ds(base, TPW)], idxb_v)
    for c in range(TPW // CCH):
        cpA = pltpu.async_copy(
            outs_hbm.at[idxa_v.at[pl.ds(c * CCH, CCH)]], bufa_v, sema)
        cpB = pltpu.async_copy(
            outs_hbm.at[idxb_v.at[pl.ds(c * CCH, CCH)]], bufb_v, semb)
        cpA.wait()
        cpB.wait()
        for r in range(CCH):
            def add_row(i, carry, r=r):
                sl = pl.ds(i * NL, NL)
                bufa_v[r, sl] = bufa_v[r, sl] + bufb_v[r, sl]
                return carry
            lax.fori_loop(0, H // NL, add_row, 0)
        pltpu.sync_copy(bufa_v, y_hbm.at[pl.ds(base + c * CCH, CCH)])


@functools.cache
def _sc_combine():
    return functools.partial(
        pl.kernel,
        out_type=jax.ShapeDtypeStruct((T, H), jnp.float32),
        mesh=plsc.VectorSubcoreMesh(core_axis_name="c", subcore_axis_name="s"),
        scratch_types=[
            pltpu.VMEM((TPW,), jnp.int32),
            pltpu.VMEM((TPW,), jnp.int32),
            pltpu.VMEM((CCH, H), jnp.float32),
            pltpu.VMEM((CCH, H), jnp.float32),
            pltpu.SemaphoreType.DMA,
            pltpu.SemaphoreType.DMA,
        ],
    )(_sc_combine_body)


def _ffn_body(be_ref, xs_ref, w1_ref, b1_ref, a1_ref, bb1_ref,
              w2_ref, b2_ref, a2_ref, bb2_ref, wrow_ref, out_ref):
    xb = xs_ref[...]
    gu = jnp.dot(xb, w1_ref[0], preferred_element_type=jnp.float32)
    lo = jnp.dot(xb, a1_ref[0], preferred_element_type=jnp.float32)
    gu = gu + jnp.dot(lo, bb1_ref[0],
                      preferred_element_type=jnp.float32) * SCALING
    gu = gu + b1_ref[0]
    gate = gu[:, :FDIM]
    up = gu[:, FDIM:]
    gate = jnp.minimum(gate, LIMIT)
    up = jnp.clip(up, -LIMIT, LIMIT)
    glu = gate * jax.nn.sigmoid(gate * ACT_ALPHA)
    gated = (up + 1.0) * glu
    dn = jnp.dot(gated, w2_ref[0], preferred_element_type=jnp.float32)
    lo2 = jnp.dot(gated, a2_ref[0], preferred_element_type=jnp.float32)
    dn = dn + jnp.dot(lo2, bb2_ref[0],
                      preferred_element_type=jnp.float32) * SCALING
    dn = dn + b2_ref[0]
    out_ref[...] = dn * wrow_ref[...]


def _ffn(xs, w1, b1, a1, bb1, w2, b2, a2, bb2, wrow, block_expert):
    b1 = b1.reshape(E, 1, D)
    b2 = b2.reshape(E, 1, H)

    def xmap(i, be):
        return (i, 0)

    def emap(i, be):
        return (be[i], 0, 0)

    grid_spec = pltpu.PrefetchScalarGridSpec(
        num_scalar_prefetch=1,
        grid=(NB,),
        in_specs=[
            pl.BlockSpec((BM, H), xmap),
            pl.BlockSpec((1, H, D), emap),
            pl.BlockSpec((1, 1, D), emap),
            pl.BlockSpec((1, H, R), emap),
            pl.BlockSpec((1, R, D), emap),
            pl.BlockSpec((1, FDIM, H), emap),
            pl.BlockSpec((1, 1, H), emap),
            pl.BlockSpec((1, FDIM, R), emap),
            pl.BlockSpec((1, R, H), emap),
            pl.BlockSpec((BM, 1), xmap),
        ],
        out_specs=pl.BlockSpec((BM, H), xmap),
    )
    return pl.pallas_call(
        _ffn_body,
        grid_spec=grid_spec,
        out_shape=jax.ShapeDtypeStruct((PM, H), jnp.float32),
    )(block_expert, xs, w1, b1, a1, bb1, w2, b2, a2, bb2, wrow)


def kernel(hidden_states, router_indices, routing_weights, gate_up_proj,
           gate_up_proj_bias, down_proj, down_proj_bias, lora_gate_up_A,
           lora_gate_up_B, lora_down_A, lora_down_B):
    batch_size = hidden_states.shape[0]
    x = hidden_states.reshape(T, H)

    ri = router_indices.reshape(-1).astype(jnp.int32)
    rw = routing_weights.reshape(-1).astype(jnp.float32)
    perm = jnp.argsort(ri).astype(jnp.int32)
    se = jnp.take(ri, perm)
    counts = jnp.zeros((E,), jnp.int32).at[ri].add(1)
    nblk = (counts + BM - 1) // BM
    pad_off = jnp.concatenate(
        [jnp.zeros((1,), jnp.int32), jnp.cumsum(nblk * BM)[:-1]])
    tight_off = jnp.concatenate(
        [jnp.zeros((1,), jnp.int32), jnp.cumsum(counts)[:-1]])
    ranks = jnp.arange(P, dtype=jnp.int32) - jnp.take(tight_off, se)
    slot = jnp.take(pad_off, se) + ranks
    tok_for_slot = jnp.zeros((PM,), jnp.int32).at[slot].set(
        (perm // TOPK).astype(jnp.int32))
    w_for_slot = jnp.zeros((PM,), jnp.float32).at[slot].set(jnp.take(rw, perm))
    pos = jnp.zeros((P,), jnp.int32).at[perm].set(slot)
    posA = pos[0::2]
    posB = pos[1::2]
    block_expert = jnp.minimum(
        jnp.searchsorted(jnp.cumsum(nblk), jnp.arange(NB), side="right"),
        E - 1).astype(jnp.int32)

    w1d = jnp.concatenate(
        [gate_up_proj[..., 0::2], gate_up_proj[..., 1::2]], axis=-1)
    b1d = jnp.concatenate(
        [gate_up_proj_bias[..., 0::2], gate_up_proj_bias[..., 1::2]], axis=-1)
    bb1d = jnp.concatenate(
        [lora_gate_up_B[..., 0::2], lora_gate_up_B[..., 1::2]], axis=-1)

    xs = _sc_gather()(x, tok_for_slot)
    outs = _ffn(xs, w1d, b1d, lora_gate_up_A,
                bb1d, down_proj, down_proj_bias, lora_down_A,
                lora_down_B, w_for_slot.reshape(PM, 1), block_expert)
    y = _sc_combine()(outs, posA, posB)
    return y.reshape(batch_size, -1, H)


# speedup vs baseline: 23.4051x; 6.0973x over previous
import functools

import jax
import jax.numpy as jnp
from jax import lax
from jax.experimental import pallas as pl
from jax.experimental.pallas import tpu as pltpu
from jax.experimental.pallas import tpu_sc as plsc

E = 16
TOPK = 2
H = 1024
FDIM = 1024
D = 2 * FDIM
R = 4
SCALING = 1.0 / R
LIMIT = 7.0
ACT_ALPHA = 1.702
T = 4096
P = T * TOPK

BM = 128
NB = P // BM + E
PM = NB * BM

NC, NS, NL = 2, 16, 16
NW = NC * NS

GPW = PM // NW
GCH = 64


def _sc_gather_body(x_hbm, tok_hbm, xs_hbm, idx_v, buf_v, sem):
    wid = lax.axis_index("s") * NC + lax.axis_index("c")
    base = wid * GPW
    pltpu.sync_copy(tok_hbm.at[pl.ds(base, GPW)], idx_v)
    for c in range(GPW // GCH):
        pltpu.async_copy(
            x_hbm.at[idx_v.at[pl.ds(c * GCH, GCH)]], buf_v, sem).wait()
        pltpu.sync_copy(buf_v, xs_hbm.at[pl.ds(base + c * GCH, GCH)])


@functools.cache
def _sc_gather():
    return functools.partial(
        pl.kernel,
        out_type=jax.ShapeDtypeStruct((PM, H), jnp.float32),
        mesh=plsc.VectorSubcoreMesh(core_axis_name="c", subcore_axis_name="s"),
        scratch_types=[
            pltpu.VMEM((GPW,), jnp.int32),
            pltpu.VMEM((GCH, H), jnp.float32),
            pltpu.SemaphoreType.DMA,
        ],
    )(_sc_gather_body)


TPW = T // NW
CCH = 32


def _sc_combine_body(outs_hbm, posa_hbm, posb_hbm, y_hbm,
                     idxa_v, idxb_v, bufa_v, bufb_v, sema, semb):
    wid = lax.axis_index("s") * NC + lax.axis_index("c")
    base = wid * TPW
    pltpu.sync_copy(posa_hbm.at[pl.ds(base, TPW)], idxa_v)
    pltpu.sync_copy(posb_hbm.at[pl.ds(base, TPW)], idxb_v)
    for c in range(TPW // CCH):
        cpA = pltpu.async_copy(
            outs_hbm.at[idxa_v.at[pl.ds(c * CCH, CCH)]], bufa_v, sema)
        cpB = pltpu.async_copy(
            outs_hbm.at[idxb_v.at[pl.ds(c * CCH, CCH)]], bufb_v, semb)
        cpA.wait()
        cpB.wait()
        for r in range(CCH):
            def add_row(i, carry, r=r):
                sl = pl.ds(i * NL, NL)
                bufa_v[r, sl] = bufa_v[r, sl] + bufb_v[r, sl]
                return carry
            lax.fori_loop(0, H // NL, add_row, 0)
        pltpu.sync_copy(bufa_v, y_hbm.at[pl.ds(base + c * CCH, CCH)])


@functools.cache
def _sc_combine():
    return functools.partial(
        pl.kernel,
        out_type=jax.ShapeDtypeStruct((T, H), jnp.float32),
        mesh=plsc.VectorSubcoreMesh(core_axis_name="c", subcore_axis_name="s"),
        scratch_types=[
            pltpu.VMEM((TPW,), jnp.int32),
            pltpu.VMEM((TPW,), jnp.int32),
            pltpu.VMEM((CCH, H), jnp.float32),
            pltpu.VMEM((CCH, H), jnp.float32),
            pltpu.SemaphoreType.DMA,
            pltpu.SemaphoreType.DMA,
        ],
    )(_sc_combine_body)


def _ffn_body(be_ref, xs_ref, w1_ref, b1_ref, a1_ref, bb1_ref,
              w2_ref, b2_ref, a2_ref, bb2_ref, wrow_ref, out_ref):
    xb = xs_ref[...]
    gu = jnp.dot(xb, w1_ref[0], preferred_element_type=jnp.float32)
    lo = jnp.dot(xb, a1_ref[0], preferred_element_type=jnp.float32)
    gu = gu + jnp.dot(lo, bb1_ref[0],
                      preferred_element_type=jnp.float32) * SCALING
    gu = gu + b1_ref[0]
    gate = gu[:, :FDIM]
    up = gu[:, FDIM:]
    gate = jnp.minimum(gate, LIMIT)
    up = jnp.clip(up, -LIMIT, LIMIT)
    glu = gate * jax.nn.sigmoid(gate * ACT_ALPHA)
    gated = (up + 1.0) * glu
    dn = jnp.dot(gated, w2_ref[0], preferred_element_type=jnp.float32)
    lo2 = jnp.dot(gated, a2_ref[0], preferred_element_type=jnp.float32)
    dn = dn + jnp.dot(lo2, bb2_ref[0],
                      preferred_element_type=jnp.float32) * SCALING
    dn = dn + b2_ref[0]
    out_ref[...] = dn * wrow_ref[...]


def _ffn(xs, w1, b1, a1, bb1, w2, b2, a2, bb2, wrow, block_expert):
    b1 = b1.reshape(E, 1, D)
    b2 = b2.reshape(E, 1, H)

    def xmap(i, be):
        return (i, 0)

    def emap(i, be):
        return (be[i], 0, 0)

    grid_spec = pltpu.PrefetchScalarGridSpec(
        num_scalar_prefetch=1,
        grid=(NB,),
        in_specs=[
            pl.BlockSpec((BM, H), xmap),
            pl.BlockSpec((1, H, D), emap),
            pl.BlockSpec((1, 1, D), emap),
            pl.BlockSpec((1, H, R), emap),
            pl.BlockSpec((1, R, D), emap),
            pl.BlockSpec((1, FDIM, H), emap),
            pl.BlockSpec((1, 1, H), emap),
            pl.BlockSpec((1, FDIM, R), emap),
            pl.BlockSpec((1, R, H), emap),
            pl.BlockSpec((BM, 1), xmap),
        ],
        out_specs=pl.BlockSpec((BM, H), xmap),
    )
    return pl.pallas_call(
        _ffn_body,
        grid_spec=grid_spec,
        out_shape=jax.ShapeDtypeStruct((PM, H), jnp.float32),
    )(block_expert, xs, w1, b1, a1, bb1, w2, b2, a2, bb2, wrow)


def kernel(hidden_states, router_indices, routing_weights, gate_up_proj,
           gate_up_proj_bias, down_proj, down_proj_bias, lora_gate_up_A,
           lora_gate_up_B, lora_down_A, lora_down_B):
    batch_size = hidden_states.shape[0]
    x = hidden_states.reshape(T, H)

    ri = router_indices.reshape(-1).astype(jnp.int32)
    rw = routing_weights.reshape(-1).astype(jnp.float32)
    perm = jnp.argsort(ri).astype(jnp.int32)
    se = jnp.take(ri, perm)
    counts = jnp.zeros((E,), jnp.int32).at[ri].add(1)
    nblk = (counts + BM - 1) // BM
    pad_off = jnp.concatenate(
        [jnp.zeros((1,), jnp.int32), jnp.cumsum(nblk * BM)[:-1]])
    tight_off = jnp.concatenate(
        [jnp.zeros((1,), jnp.int32), jnp.cumsum(counts)[:-1]])
    ranks = jnp.arange(P, dtype=jnp.int32) - jnp.take(tight_off, se)
    slot = jnp.take(pad_off, se) + ranks
    tok_for_slot = jnp.zeros((PM,), jnp.int32).at[slot].set(
        (perm // TOPK).astype(jnp.int32))
    w_for_slot = jnp.zeros((PM,), jnp.float32).at[slot].set(jnp.take(rw, perm))
    pos = jnp.zeros((P,), jnp.int32).at[perm].set(slot)
    posA = pos[0::2]
    posB = pos[1::2]
    block_expert = jnp.minimum(
        jnp.searchsorted(jnp.cumsum(nblk), jnp.arange(NB), side="right"),
        E - 1).astype(jnp.int32)

    w1d = gate_up_proj.reshape(E, H, FDIM, 2).swapaxes(-1, -2).reshape(E, H, D)
    b1d = gate_up_proj_bias.reshape(E, FDIM, 2).swapaxes(-1, -2).reshape(E, D)
    bb1d = lora_gate_up_B.reshape(E, R, FDIM, 2).swapaxes(-1, -2).reshape(E, R, D)

    xs = _sc_gather()(x, tok_for_slot)
    outs = _ffn(xs, w1d, b1d, lora_gate_up_A,
                bb1d, down_proj, down_proj_bias, lora_down_A,
                lora_down_B, w_for_slot.reshape(PM, 1), block_expert)
    y = _sc_combine()(outs, posA, posB)
    return y.reshape(batch_size, -1, H)


# speedup vs baseline: 32.2701x; 1.3788x over previous
import functools

import jax
import jax.numpy as jnp
import numpy as np
from jax import lax
from jax.experimental import pallas as pl
from jax.experimental.pallas import tpu as pltpu
from jax.experimental.pallas import tpu_sc as plsc

E = 16
TOPK = 2
H = 1024
FDIM = 1024
D = 2 * FDIM
R = 4
SCALING = 1.0 / R
LIMIT = 7.0
ACT_ALPHA = 1.702
T = 4096
P = T * TOPK

BM = 128
NB = P // BM + E
PM = NB * BM

NC, NS, NL = 2, 16, 16
NW = NC * NS

GPW = PM // NW
GCH = 64


def _sc_gather_body(x_hbm, tok_hbm, xs_hbm, idx_v, buf_v, sem):
    wid = lax.axis_index("s") * NC + lax.axis_index("c")
    base = wid * GPW
    pltpu.sync_copy(tok_hbm.at[pl.ds(base, GPW)], idx_v)
    for c in range(GPW // GCH):
        pltpu.async_copy(
            x_hbm.at[idx_v.at[pl.ds(c * GCH, GCH)]], buf_v, sem).wait()
        pltpu.sync_copy(buf_v, xs_hbm.at[pl.ds(base + c * GCH, GCH)])


@functools.cache
def _sc_gather():
    return functools.partial(
        pl.kernel,
        out_type=jax.ShapeDtypeStruct((PM, H), jnp.float32),
        mesh=plsc.VectorSubcoreMesh(core_axis_name="c", subcore_axis_name="s"),
        scratch_types=[
            pltpu.VMEM((GPW,), jnp.int32),
            pltpu.VMEM((GCH, H), jnp.float32),
            pltpu.SemaphoreType.DMA,
        ],
    )(_sc_gather_body)


TPW = T // NW
CCH = 32


def _sc_combine_body(outs_hbm, posa_hbm, posb_hbm, y_hbm,
                     idxa_v, idxb_v, bufa_v, bufb_v, sema, semb):
    wid = lax.axis_index("s") * NC + lax.axis_index("c")
    base = wid * TPW
    pltpu.sync_copy(posa_hbm.at[pl.ds(base, TPW)], idxa_v)
    pltpu.sync_copy(posb_hbm.at[pl.ds(base, TPW)], idxb_v)
    for c in range(TPW // CCH):
        cpA = pltpu.async_copy(
            outs_hbm.at[idxa_v.at[pl.ds(c * CCH, CCH)]], bufa_v, sema)
        cpB = pltpu.async_copy(
            outs_hbm.at[idxb_v.at[pl.ds(c * CCH, CCH)]], bufb_v, semb)
        cpA.wait()
        cpB.wait()
        for r in range(CCH):
            def add_row(i, carry, r=r):
                sl = pl.ds(i * NL, NL)
                bufa_v[r, sl] = bufa_v[r, sl] + bufb_v[r, sl]
                return carry
            lax.fori_loop(0, H // NL, add_row, 0)
        pltpu.sync_copy(bufa_v, y_hbm.at[pl.ds(base + c * CCH, CCH)])


@functools.cache
def _sc_combine():
    return functools.partial(
        pl.kernel,
        out_type=jax.ShapeDtypeStruct((T, H), jnp.float32),
        mesh=plsc.VectorSubcoreMesh(core_axis_name="c", subcore_axis_name="s"),
        scratch_types=[
            pltpu.VMEM((TPW,), jnp.int32),
            pltpu.VMEM((TPW,), jnp.int32),
            pltpu.VMEM((CCH, H), jnp.float32),
            pltpu.VMEM((CCH, H), jnp.float32),
            pltpu.SemaphoreType.DMA,
            pltpu.SemaphoreType.DMA,
        ],
    )(_sc_combine_body)


def _ffn_body(be_ref, xs_ref, w1_ref, b1_ref, a1_ref, bb1_ref,
              w2_ref, b2_ref, a2_ref, bb2_ref, wrow_ref, p_ref, out_ref):
    xb = xs_ref[...]
    gu = jnp.dot(xb, w1_ref[0], preferred_element_type=jnp.float32)
    lo = jnp.dot(xb, a1_ref[0], preferred_element_type=jnp.float32)
    gu = gu + jnp.dot(lo, bb1_ref[0],
                      preferred_element_type=jnp.float32) * SCALING
    gu = gu + b1_ref[0]
    even = jax.lax.broadcasted_iota(jnp.int32, (BM, D), 1) % 2 == 0
    glu_b = jnp.minimum(gu, LIMIT)
    glu_b = glu_b * jax.nn.sigmoid(glu_b * ACT_ALPHA)
    up_b = jnp.clip(gu, -LIMIT, LIMIT) + 1.0
    z = jnp.where(even, glu_b, up_b)
    v = jnp.where(even, z * pltpu.roll(z, D - 1, 1), 0.0)
    gated = jnp.dot(v, p_ref[...], preferred_element_type=jnp.float32)
    dn = jnp.dot(gated, w2_ref[0], preferred_element_type=jnp.float32)
    lo2 = jnp.dot(gated, a2_ref[0], preferred_element_type=jnp.float32)
    dn = dn + jnp.dot(lo2, bb2_ref[0],
                      preferred_element_type=jnp.float32) * SCALING
    dn = dn + b2_ref[0]
    out_ref[...] = dn * wrow_ref[...]


_P_SEL = np.zeros((D, FDIM), dtype=np.float32)
_P_SEL[2 * np.arange(FDIM), np.arange(FDIM)] = 1.0


def _ffn(xs, w1, b1, a1, bb1, w2, b2, a2, bb2, wrow, block_expert):
    b1 = b1.reshape(E, 1, D)
    b2 = b2.reshape(E, 1, H)

    def xmap(i, be):
        return (i, 0)

    def emap(i, be):
        return (be[i], 0, 0)

    def cmap(i, be):
        return (0, 0)

    grid_spec = pltpu.PrefetchScalarGridSpec(
        num_scalar_prefetch=1,
        grid=(NB,),
        in_specs=[
            pl.BlockSpec((BM, H), xmap),
            pl.BlockSpec((1, H, D), emap),
            pl.BlockSpec((1, 1, D), emap),
            pl.BlockSpec((1, H, R), emap),
            pl.BlockSpec((1, R, D), emap),
            pl.BlockSpec((1, FDIM, H), emap),
            pl.BlockSpec((1, 1, H), emap),
            pl.BlockSpec((1, FDIM, R), emap),
            pl.BlockSpec((1, R, H), emap),
            pl.BlockSpec((BM, 1), xmap),
            pl.BlockSpec((D, FDIM), cmap),
        ],
        out_specs=pl.BlockSpec((BM, H), xmap),
    )
    return pl.pallas_call(
        _ffn_body,
        grid_spec=grid_spec,
        out_shape=jax.ShapeDtypeStruct((PM, H), jnp.float32),
    )(block_expert, xs, w1, b1, a1, bb1, w2, b2, a2, bb2, wrow,
      jnp.asarray(_P_SEL))


def kernel(hidden_states, router_indices, routing_weights, gate_up_proj,
           gate_up_proj_bias, down_proj, down_proj_bias, lora_gate_up_A,
           lora_gate_up_B, lora_down_A, lora_down_B):
    batch_size = hidden_states.shape[0]
    x = hidden_states.reshape(T, H)

    ri = router_indices.reshape(-1).astype(jnp.int32)
    rw = routing_weights.reshape(-1).astype(jnp.float32)
    perm = jnp.argsort(ri).astype(jnp.int32)
    se = jnp.take(ri, perm)
    counts = jnp.zeros((E,), jnp.int32).at[ri].add(1)
    nblk = (counts + BM - 1) // BM
    pad_off = jnp.concatenate(
        [jnp.zeros((1,), jnp.int32), jnp.cumsum(nblk * BM)[:-1]])
    tight_off = jnp.concatenate(
        [jnp.zeros((1,), jnp.int32), jnp.cumsum(counts)[:-1]])
    ranks = jnp.arange(P, dtype=jnp.int32) - jnp.take(tight_off, se)
    slot = jnp.take(pad_off, se) + ranks
    tok_for_slot = jnp.zeros((PM,), jnp.int32).at[slot].set(
        (perm // TOPK).astype(jnp.int32))
    w_for_slot = jnp.zeros((PM,), jnp.float32).at[slot].set(jnp.take(rw, perm))
    pos = jnp.zeros((P,), jnp.int32).at[perm].set(slot)
    posA = pos[0::2]
    posB = pos[1::2]
    block_expert = jnp.minimum(
        jnp.searchsorted(jnp.cumsum(nblk), jnp.arange(NB), side="right"),
        E - 1).astype(jnp.int32)

    xs = _sc_gather()(x, tok_for_slot)
    outs = _ffn(xs, gate_up_proj, gate_up_proj_bias, lora_gate_up_A,
                lora_gate_up_B, down_proj, down_proj_bias, lora_down_A,
                lora_down_B, w_for_slot.reshape(PM, 1), block_expert)
    y = _sc_combine()(outs, posA, posB)
    return y.reshape(batch_size, -1, H)


# speedup vs baseline: 32.8112x; 1.0168x over previous
import functools

import jax
import jax.numpy as jnp
import numpy as np
from jax import lax
from jax.experimental import pallas as pl
from jax.experimental.pallas import tpu as pltpu
from jax.experimental.pallas import tpu_sc as plsc

E = 16
TOPK = 2
H = 1024
FDIM = 1024
D = 2 * FDIM
R = 4
SCALING = 1.0 / R
LIMIT = 7.0
ACT_ALPHA = 1.702
T = 4096
P = T * TOPK

BM = 128
NB = P // BM + E
PM = NB * BM

NC, NS, NL = 2, 16, 16
NW = NC * NS

GPW = PM // NW
GCH = 40
GNC = GPW // GCH


def _sc_gather_body(x_hbm, tok_hbm, xs_hbm, idx_v, buf0_v, buf1_v,
                    gsem0, gsem1, wsem0, wsem1):
    wid = lax.axis_index("s") * NC + lax.axis_index("c")
    base = wid * GPW
    bufs = (buf0_v, buf1_v)
    gsems = (gsem0, gsem1)
    wsems = (wsem0, wsem1)
    pltpu.sync_copy(tok_hbm.at[pl.ds(base, GPW)], idx_v)
    gets = [None, None]
    puts = [None, None]
    gets[0] = pltpu.async_copy(
        x_hbm.at[idx_v.at[pl.ds(0, GCH)]], bufs[0], gsems[0])
    for c in range(GNC):
        b = c % 2
        nb = (c + 1) % 2
        if c + 1 < GNC:
            if puts[nb] is not None:
                puts[nb].wait()
            gets[nb] = pltpu.async_copy(
                x_hbm.at[idx_v.at[pl.ds((c + 1) * GCH, GCH)]],
                bufs[nb], gsems[nb])
        gets[b].wait()
        puts[b] = pltpu.async_copy(
            bufs[b], xs_hbm.at[pl.ds(base + c * GCH, GCH)], wsems[b])
    puts[0].wait()
    puts[1].wait()


@functools.cache
def _sc_gather():
    return functools.partial(
        pl.kernel,
        out_type=jax.ShapeDtypeStruct((PM, H), jnp.float32),
        mesh=plsc.VectorSubcoreMesh(core_axis_name="c", subcore_axis_name="s"),
        scratch_types=[
            pltpu.VMEM((GPW,), jnp.int32),
            pltpu.VMEM((GCH, H), jnp.float32),
            pltpu.VMEM((GCH, H), jnp.float32),
            pltpu.SemaphoreType.DMA,
            pltpu.SemaphoreType.DMA,
            pltpu.SemaphoreType.DMA,
            pltpu.SemaphoreType.DMA,
        ],
    )(_sc_gather_body)


TPW = T // NW
CCH = 16
CNC = TPW // CCH


def _sc_combine_body(outs_hbm, posa_hbm, posb_hbm, y_hbm,
                     idxa_v, idxb_v, a0_v, b0_v, a1_v, b1_v,
                     sa0, sb0, sa1, sb1, w0, w1):
    wid = lax.axis_index("s") * NC + lax.axis_index("c")
    base = wid * TPW
    abufs = (a0_v, a1_v)
    bbufs = (b0_v, b1_v)
    asems = (sa0, sa1)
    bsems = (sb0, sb1)
    wsems = (w0, w1)
    pltpu.sync_copy(posa_hbm.at[pl.ds(base, TPW)], idxa_v)
    pltpu.sync_copy(posb_hbm.at[pl.ds(base, TPW)], idxb_v)
    getsa = [None, None]
    getsb = [None, None]
    puts = [None, None]
    getsa[0] = pltpu.async_copy(
        outs_hbm.at[idxa_v.at[pl.ds(0, CCH)]], abufs[0], asems[0])
    getsb[0] = pltpu.async_copy(
        outs_hbm.at[idxb_v.at[pl.ds(0, CCH)]], bbufs[0], bsems[0])
    for c in range(CNC):
        b = c % 2
        nb = (c + 1) % 2
        if c + 1 < CNC:
            if puts[nb] is not None:
                puts[nb].wait()
            getsa[nb] = pltpu.async_copy(
                outs_hbm.at[idxa_v.at[pl.ds((c + 1) * CCH, CCH)]],
                abufs[nb], asems[nb])
            getsb[nb] = pltpu.async_copy(
                outs_hbm.at[idxb_v.at[pl.ds((c + 1) * CCH, CCH)]],
                bbufs[nb], bsems[nb])
        getsa[b].wait()
        getsb[b].wait()
        for r in range(CCH):
            def add_row(i, carry, r=r, ab=abufs[b], bb=bbufs[b]):
                sl = pl.ds(i * NL, NL)
                ab[r, sl] = ab[r, sl] + bb[r, sl]
                return carry
            lax.fori_loop(0, H // NL, add_row, 0)
        puts[b] = pltpu.async_copy(
            abufs[b], y_hbm.at[pl.ds(base + c * CCH, CCH)], wsems[b])
    puts[0].wait()
    puts[1].wait()


@functools.cache
def _sc_combine():
    return functools.partial(
        pl.kernel,
        out_type=jax.ShapeDtypeStruct((T, H), jnp.float32),
        mesh=plsc.VectorSubcoreMesh(core_axis_name="c", subcore_axis_name="s"),
        scratch_types=[
            pltpu.VMEM((TPW,), jnp.int32),
            pltpu.VMEM((TPW,), jnp.int32),
            pltpu.VMEM((CCH, H), jnp.float32),
            pltpu.VMEM((CCH, H), jnp.float32),
            pltpu.VMEM((CCH, H), jnp.float32),
            pltpu.VMEM((CCH, H), jnp.float32),
            pltpu.SemaphoreType.DMA,
            pltpu.SemaphoreType.DMA,
            pltpu.SemaphoreType.DMA,
            pltpu.SemaphoreType.DMA,
            pltpu.SemaphoreType.DMA,
            pltpu.SemaphoreType.DMA,
        ],
    )(_sc_combine_body)


def _ffn_body(be_ref, xs_ref, w1_ref, b1_ref, a1_ref, bb1_ref,
              w2_ref, b2_ref, a2_ref, bb2_ref, wrow_ref, p_ref, out_ref):
    xb = xs_ref[...]
    gu = jnp.dot(xb, w1_ref[0], preferred_element_type=jnp.float32)
    lo = jnp.dot(xb, a1_ref[0], preferred_element_type=jnp.float32)
    gu = gu + jnp.dot(lo, bb1_ref[0],
                      preferred_element_type=jnp.float32) * SCALING
    gu = gu + b1_ref[0]
    even = jax.lax.broadcasted_iota(jnp.int32, (BM, D), 1) % 2 == 0
    glu_b = jnp.minimum(gu, LIMIT)
    glu_b = glu_b * jax.nn.sigmoid(glu_b * ACT_ALPHA)
    up_b = jnp.clip(gu, -LIMIT, LIMIT) + 1.0
    z = jnp.where(even, glu_b, up_b)
    v = jnp.where(even, z * pltpu.roll(z, D - 1, 1), 0.0)
    gated = jnp.dot(v, p_ref[...], preferred_element_type=jnp.float32)
    dn = jnp.dot(gated, w2_ref[0], preferred_element_type=jnp.float32)
    lo2 = jnp.dot(gated, a2_ref[0], preferred_element_type=jnp.float32)
    dn = dn + jnp.dot(lo2, bb2_ref[0],
                      preferred_element_type=jnp.float32) * SCALING
    dn = dn + b2_ref[0]
    out_ref[...] = dn * wrow_ref[...]


_P_SEL = np.zeros((D, FDIM), dtype=np.float32)
_P_SEL[2 * np.arange(FDIM), np.arange(FDIM)] = 1.0


def _ffn(xs, w1, b1, a1, bb1, w2, b2, a2, bb2, wrow, block_expert):
    b1 = b1.reshape(E, 1, D)
    b2 = b2.reshape(E, 1, H)

    def xmap(i, be):
        return (i, 0)

    def emap(i, be):
        return (be[i], 0, 0)

    def cmap(i, be):
        return (0, 0)

    grid_spec = pltpu.PrefetchScalarGridSpec(
        num_scalar_prefetch=1,
        grid=(NB,),
        in_specs=[
            pl.BlockSpec((BM, H), xmap),
            pl.BlockSpec((1, H, D), emap),
            pl.BlockSpec((1, 1, D), emap),
            pl.BlockSpec((1, H, R), emap),
            pl.BlockSpec((1, R, D), emap),
            pl.BlockSpec((1, FDIM, H), emap),
            pl.BlockSpec((1, 1, H), emap),
            pl.BlockSpec((1, FDIM, R), emap),
            pl.BlockSpec((1, R, H), emap),
            pl.BlockSpec((BM, 1), xmap),
            pl.BlockSpec((D, FDIM), cmap),
        ],
        out_specs=pl.BlockSpec((BM, H), xmap),
    )
    return pl.pallas_call(
        _ffn_body,
        grid_spec=grid_spec,
        out_shape=jax.ShapeDtypeStruct((PM, H), jnp.float32),
    )(block_expert, xs, w1, b1, a1, bb1, w2, b2, a2, bb2, wrow,
      jnp.asarray(_P_SEL))


def kernel(hidden_states, router_indices, routing_weights, gate_up_proj,
           gate_up_proj_bias, down_proj, down_proj_bias, lora_gate_up_A,
           lora_gate_up_B, lora_down_A, lora_down_B):
    batch_size = hidden_states.shape[0]
    x = hidden_states.reshape(T, H)

    ri = router_indices.reshape(-1).astype(jnp.int32)
    rw = routing_weights.reshape(-1).astype(jnp.float32)
    perm = jnp.argsort(ri).astype(jnp.int32)
    se = jnp.take(ri, perm)
    counts = jnp.zeros((E,), jnp.int32).at[ri].add(1)
    nblk = (counts + BM - 1) // BM
    pad_off = jnp.concatenate(
        [jnp.zeros((1,), jnp.int32), jnp.cumsum(nblk * BM)[:-1]])
    tight_off = jnp.concatenate(
        [jnp.zeros((1,), jnp.int32), jnp.cumsum(counts)[:-1]])
    ranks = jnp.arange(P, dtype=jnp.int32) - jnp.take(tight_off, se)
    slot = jnp.take(pad_off, se) + ranks
    tok_for_slot = jnp.zeros((PM,), jnp.int32).at[slot].set(
        (perm // TOPK).astype(jnp.int32))
    w_for_slot = jnp.zeros((PM,), jnp.float32).at[slot].set(jnp.take(rw, perm))
    pos = jnp.zeros((P,), jnp.int32).at[perm].set(slot)
    posA = pos[0::2]
    posB = pos[1::2]
    block_expert = jnp.minimum(
        jnp.searchsorted(jnp.cumsum(nblk), jnp.arange(NB), side="right"),
        E - 1).astype(jnp.int32)

    xs = _sc_gather()(x, tok_for_slot)
    outs = _ffn(xs, gate_up_proj, gate_up_proj_bias, lora_gate_up_A,
                lora_gate_up_B, down_proj, down_proj_bias, lora_down_A,
                lora_down_B, w_for_slot.reshape(PM, 1), block_expert)
    y = _sc_combine()(outs, posA, posB)
    return y.reshape(batch_size, -1, H)
